# Initial kernel scaffold; baseline (speedup 1.0000x reference)
import jax
import jax.numpy as jnp
from jax.experimental import pallas as pl


def kernel(x, w_enc, w_dec, w_out, bias):
    raise NotImplementedError("write your pallas kernel here")


# speedup vs baseline: 1.4341x; 1.4341x over previous
import functools

import jax
import jax.numpy as jnp
from jax.experimental import pallas as pl
from jax.experimental.pallas import tpu as pltpu

_TAU = 8
_NB = 2


def _cdiv(a, b):
    return -(-a // b)


def _gru_body(x_ref, we_ref, wd_ref, wf_ref, wo_ref, b_ref, y_ref,
              *, seq_len, tau, hp, out_lanes):
    T, HP, OUT = seq_len, hp, out_lanes
    BB = y_ref.shape[0]
    f32, bf16 = jnp.float32, jnp.bfloat16
    dot = lambda a, b: jnp.dot(a, b, preferred_element_type=f32)

    we = we_ref[...]
    wd = wd_ref[...]
    wf = wf_ref[...]
    wo = wo_ref[...]
    b = b_ref[...]

    be = jnp.broadcast_to(b[0:1, :], (BB, 4 * HP))
    bd = jnp.broadcast_to(b[1:2, :], (BB, 4 * HP))
    bf_ = jnp.broadcast_to(b[2:3, :], (BB, 4 * HP))
    bo = jnp.broadcast_to(b[3:4, 0:OUT], (BB, OUT))

    xb = x_ref[...]

    def gate_update(g, h):
        r = jax.nn.sigmoid(g[:, 0:HP])
        z = jax.nn.sigmoid(g[:, HP:2 * HP])
        n = jnp.tanh(g[:, 2 * HP:3 * HP] + r * g[:, 3 * HP:4 * HP])
        return n + z * (h - n)

    h = jnp.zeros((BB, HP), f32)
    for t in range(T):
        cat = jnp.concatenate([xb[t * BB:(t + 1) * BB, :], h.astype(bf16)],
                              axis=-1)
        h = gate_update(dot(cat, we) + be, h)

    cat = jnp.concatenate([xb[(T - 1) * BB:T * BB, :], h.astype(bf16)],
                          axis=-1)
    h = gate_update(dot(cat, wd) + bd, h)
    hb = h.astype(bf16)
    y_ref[:, 0:OUT] = dot(hb, wo) + bo

    for k in range(1, tau):
        h = gate_update(dot(hb, wf) + bf_, h)
        hb = h.astype(bf16)
        y_ref[:, k * OUT:(k + 1) * OUT] = dot(hb, wo) + bo


def kernel(x, w_enc, w_dec, w_out, bias):
    B, T, D = x.shape
    HP = w_enc.shape[0] - D
    OUT = w_out.shape[1]
    f32, bf16 = jnp.float32, jnp.bfloat16

    nb = _NB
    BB = _cdiv(_cdiv(B, nb), 8) * 8
    Bp = nb * BB

    w_enc4 = jnp.concatenate([
        w_enc[:, 0:2 * HP],
        jnp.concatenate([w_enc[:D, 2 * HP:3 * HP],
                         jnp.zeros((HP, HP), f32)], axis=0),
        jnp.concatenate([jnp.zeros((D, HP), f32),
                         w_enc[D:, 2 * HP:3 * HP]], axis=0),
    ], axis=1).astype(bf16)
    w_dec4 = w_dec.astype(bf16)
    w_decf = (jnp.dot(w_out, w_dec[:HP, :]) + w_dec[HP:, :]).astype(bf16)
    w_outb = w_out.astype(bf16)

    bslab = jnp.zeros((8, 4 * HP), f32)
    bslab = bslab.at[0:1, :].set(jnp.concatenate(
        [bias[0:1, 0:2 * HP], bias[1:2, 0:HP], bias[0:1, 2 * HP:3 * HP]],
        axis=1))
    bslab = bslab.at[1:2, :].set(bias[2:3, :])
    bslab = bslab.at[2:3, :].set(
        bias[2:3, :] + jnp.dot(bias[3:4, 0:OUT], w_dec[:HP, :]))
    bslab = bslab.at[3:4, 0:OUT].set(bias[3:4, 0:OUT])

    x_p = jnp.pad(x, ((0, Bp - B), (0, 0), (0, 0)))
    x_flat = (x_p.reshape(nb, BB, T, D).transpose(0, 2, 1, 3)
              .reshape(nb * T * BB, D).astype(bf16))

    body = functools.partial(_gru_body, seq_len=T, tau=_TAU, hp=HP,
                             out_lanes=OUT)

    y = pl.pallas_call(
        body,
        out_shape=jax.ShapeDtypeStruct((Bp, _TAU * OUT), f32),
        grid=(nb,),
        in_specs=[
            pl.BlockSpec((T * BB, D), lambda b: (b, 0)),
            pl.BlockSpec(w_enc4.shape, lambda b: (0, 0)),
            pl.BlockSpec(w_dec4.shape, lambda b: (0, 0)),
            pl.BlockSpec(w_decf.shape, lambda b: (0, 0)),
            pl.BlockSpec(w_outb.shape, lambda b: (0, 0)),
            pl.BlockSpec(bslab.shape, lambda b: (0, 0)),
        ],
        out_specs=pl.BlockSpec((BB, _TAU * OUT), lambda b: (b, 0)),
        compiler_params=pltpu.CompilerParams(
            dimension_semantics=("parallel",)),
    )(x_flat, w_enc4, w_dec4, w_decf, w_outb, bslab)

    return y.reshape(Bp, _TAU, OUT)[:B, :, :D]
